# Initial kernel scaffold; baseline (speedup 1.0000x reference)
import jax
import jax.numpy as jnp
from jax.experimental import pallas as pl


def kernel(x, edge_index, edge_attr, available, params):
    raise NotImplementedError("write your pallas kernel here")


# speedup vs baseline: 8.3473x; 8.3473x over previous
import functools

import jax
import jax.numpy as jnp
from jax import lax
from jax.experimental import pallas as pl
from jax.experimental.pallas import tpu as pltpu
from jax.experimental.pallas import tpu_sc as plsc

NN = 10000
EE = 320000
LAT = 64
HID = 128
NPAD = 10112
EPAD = 327680
NC, NS = 2, 16
NW = NC * NS
EW = EPAD // NW
NB = EW // 128
NROWS = NPAD // NS

_HI = lax.Precision.HIGHEST


def _edge_body(e_ref, we_ref, a2_ref, en_ref, se_ref, semax_ref):
    et = jnp.dot(e_ref[...], we_ref[...], precision=_HI,
                 preferred_element_type=jnp.float32)
    en_ref[...] = jnp.where(et > 0, et, (jnp.exp(et) - 1.0))
    sev = jnp.dot(et, a2_ref[...], precision=_HI,
                  preferred_element_type=jnp.float32)
    se_ref[...] = sev
    i = pl.program_id(0)
    prev = jnp.where(i == 0, jnp.full((1, 1), -jnp.inf), semax_ref[...])
    semax_ref[...] = jnp.maximum(prev, jnp.full((1, 1), jnp.max(sev)))


def _edge_call(e, We, a2):
    rb = 2048
    de = e.shape[1]
    grid = EPAD // rb
    return pl.pallas_call(
        _edge_body,
        grid=(grid,),
        in_specs=[
            pl.BlockSpec((rb, de), lambda i: (i, 0)),
            pl.BlockSpec((de, LAT), lambda i: (0, 0)),
            pl.BlockSpec((LAT, 1), lambda i: (0, 0)),
        ],
        out_specs=[
            pl.BlockSpec((rb, LAT), lambda i: (i, 0)),
            pl.BlockSpec((rb, 1), lambda i: (i, 0)),
            pl.BlockSpec((1, 1), lambda i: (0, 0)),
        ],
        out_shape=[
            jax.ShapeDtypeStruct((EPAD, LAT), jnp.float32),
            jax.ShapeDtypeStruct((EPAD, 1), jnp.float32),
            jax.ShapeDtypeStruct((1, 1), jnp.float32),
        ],
    )(e, We, a2)


NBLK = 1264


def _node_tail(h, wn_ref, a01_ref, hslo_ref, hshi_ref, s0_ref, s1_ref,
               s0max_ref):
    hs = jnp.dot(h, wn_ref[...], precision=_HI,
                 preferred_element_type=jnp.float32)
    hslo_ref[...] = hs[:, :LAT // 2]
    hshi_ref[...] = hs[:, LAT // 2:]
    s = jnp.dot(hs, a01_ref[...], precision=_HI,
                preferred_element_type=jnp.float32)
    s0_ref[...] = s[:, 0:1]
    s1_ref[...] = s[:, 1:2]
    i = pl.program_id(0)
    prev = jnp.where(i == 0, jnp.full((1, 1), -jnp.inf), s0max_ref[...])
    s0max_ref[...] = jnp.maximum(prev, jnp.full((1, 1), jnp.max(s[:, 0])))


def _node0_body(x_ref, wn_ref, a01_ref, hslo_ref, hshi_ref, s0_ref, s1_ref,
                s0max_ref):
    _node_tail(x_ref[...], wn_ref, a01_ref, hslo_ref, hshi_ref, s0_ref,
               s1_ref, s0max_ref)


def _nodec_body(plo_ref, phi_ref, den_ref, wn_ref, a01_ref, hslo_ref,
                hshi_ref, s0_ref, s1_ref, s0max_ref):
    den = den_ref[0] + den_ref[1] + 1e-9
    o = jnp.concatenate([plo_ref[0] + plo_ref[1], phi_ref[0] + phi_ref[1]],
                        axis=1) / den
    h = jnp.where(o > 0, o, (jnp.exp(o) - 1.0))
    _node_tail(h, wn_ref, a01_ref, hslo_ref, hshi_ref, s0_ref, s1_ref,
               s0max_ref)


_NODE_OUT = [
    jax.ShapeDtypeStruct((NPAD, LAT // 2), jnp.float32),
    jax.ShapeDtypeStruct((NPAD, LAT // 2), jnp.float32),
    jax.ShapeDtypeStruct((NPAD, 1), jnp.float32),
    jax.ShapeDtypeStruct((NPAD, 1), jnp.float32),
    jax.ShapeDtypeStruct((1, 1), jnp.float32),
]

_NODE_OUT_SPECS = [
    pl.BlockSpec((NBLK, LAT // 2), lambda i: (i, 0)),
    pl.BlockSpec((NBLK, LAT // 2), lambda i: (i, 0)),
    pl.BlockSpec((NBLK, 1), lambda i: (i, 0)),
    pl.BlockSpec((NBLK, 1), lambda i: (i, 0)),
    pl.BlockSpec((1, 1), lambda i: (0, 0)),
]


def _node0_call(x, Wn, a01):
    dn = x.shape[1]
    return pl.pallas_call(
        _node0_body,
        grid=(NPAD // NBLK,),
        in_specs=[
            pl.BlockSpec((NBLK, dn), lambda i: (i, 0)),
            pl.BlockSpec((dn, LAT), lambda i: (0, 0)),
            pl.BlockSpec((LAT, 2), lambda i: (0, 0)),
        ],
        out_specs=_NODE_OUT_SPECS,
        out_shape=_NODE_OUT,
    )(x, Wn, a01)


def _nodec_call(plo, phi, den, Wn, a01):
    return pl.pallas_call(
        _nodec_body,
        grid=(NPAD // NBLK,),
        in_specs=[
            pl.BlockSpec((NC, NBLK, LAT // 2), lambda i: (0, i, 0)),
            pl.BlockSpec((NC, NBLK, LAT // 2), lambda i: (0, i, 0)),
            pl.BlockSpec((NC, NBLK, 1), lambda i: (0, i, 0)),
            pl.BlockSpec((LAT, LAT), lambda i: (0, 0)),
            pl.BlockSpec((LAT, 2), lambda i: (0, 0)),
        ],
        out_specs=_NODE_OUT_SPECS,
        out_shape=_NODE_OUT,
    )(plo, phi, den, Wn, a01)


def _sc_body(hslo_hbm, hshi_hbm, s0_hbm, s1_hbm, m_hbm, se_hbm, src_hbm,
             dst_hbm, dst2d_hbm, outlo_hbm, outhi_hbm, den_hbm, s0_v, s1_v,
             m_v, se_v, src_v, dst_v, dst_m, ex_v, rows_v, zden_v,
             shared_out, shared_den, sem):
    cid = lax.axis_index("c")
    sid = lax.axis_index("s")
    wid = sid * NC + cid
    ebase = wid * EW
    rbase = wid * NB
    HLAT = LAT // 2

    pltpu.sync_copy(s0_hbm, s0_v)
    pltpu.sync_copy(s1_hbm, s1_v)
    pltpu.sync_copy(m_hbm, m_v)
    pltpu.sync_copy(se_hbm.at[pl.ds(ebase, EW)], se_v)
    pltpu.sync_copy(src_hbm.at[pl.ds(ebase, EW)], src_v)
    pltpu.sync_copy(dst_hbm.at[pl.ds(ebase, EW)], dst_v)
    pltpu.sync_copy(dst2d_hbm.at[pl.ds(rbase, NB)], dst_m)

    zeros16 = jnp.zeros((16,), jnp.float32)

    def zrow(r, _):
        for c in range(HLAT // 16):
            rows_v[r, pl.ds(c * 16, 16)] = zeros16
        return 0

    lax.fori_loop(0, 128, zrow, 0)

    def zden(r, _):
        zden_v[pl.ds(r * 16, 16)] = zeros16
        return 0

    lax.fori_loop(0, 640 // 16, zden, 0)

    for half in range(2):
        hs_hbm = hslo_hbm if half == 0 else hshi_hbm
        out_hbm = outlo_hbm if half == 0 else outhi_hbm

        for r in range(NROWS // 128):
            pltpu.sync_copy(
                rows_v, shared_out.at[pl.ds(sid * NROWS + r * 128, 128)])
        _rem = NROWS % 128
        if _rem:
            pltpu.sync_copy(
                rows_v.at[pl.ds(0, _rem)],
                shared_out.at[pl.ds(sid * NROWS + (NROWS // 128) * 128,
                                    _rem)])
        if half == 0:
            pltpu.sync_copy(zden_v.at[pl.ds(0, NROWS)],
                            shared_den.at[pl.ds(sid * NROWS, NROWS)])
        plsc.subcore_barrier()

        def batch(j, _):
            if half == 0:
                def grp(i, carry):
                    b = j * 128 + i * 16
                    s16 = src_v[pl.ds(b, 16)]
                    d16 = dst_v[pl.ds(b, 16)]
                    g0 = plsc.load_gather(s0_v, [s16])
                    g1 = plsc.load_gather(s1_v, [d16])
                    lg = g0 + g1 + se_v[pl.ds(b, 16)]
                    lg = jnp.maximum(lg, lg * 0.2)
                    pre = m_v[...] + g1
                    gt = jnp.maximum(pre, pre * 0.2)
                    ex_v[pl.ds(b, 16)] = jnp.exp(lg - gt)
                    return carry

                lax.fori_loop(0, 8, grp, 0)

            pltpu.async_copy(hs_hbm.at[src_v.at[pl.ds(j * 128, 128)]],
                             rows_v, sem).wait()

            def scale16(i, carry):
                ex16 = ex_v[pl.ds(j * 128 + i * 16, 16)]
                for e in range(16):
                    a = ex16[e]
                    r = i * 16 + e
                    for c in range(HLAT // 16):
                        sl = pl.ds(c * 16, 16)
                        rows_v[r, sl] = rows_v[r, sl] * a
                return carry

            lax.fori_loop(0, 8, scale16, 0)

            pltpu.sync_copy(rows_v, shared_out.at[dst_m.at[j]], add=True)
            if half == 0:
                pltpu.sync_copy(ex_v.at[pl.ds(j * 128, 128)],
                                shared_den.at[dst_m.at[j]], add=True)
            return 0

        lax.fori_loop(0, NB, batch, 0)
        plsc.subcore_barrier()

        pltpu.sync_copy(shared_out.at[pl.ds(sid * NROWS, NROWS)],
                        out_hbm.at[cid, pl.ds(sid * NROWS, NROWS)])
        if half == 0:
            pltpu.sync_copy(shared_den.at[pl.ds(sid * NROWS, NROWS)],
                            den_hbm.at[cid, pl.ds(sid * NROWS, NROWS)])


_sc_call = functools.partial(
    pl.kernel,
    out_type=(
        jax.ShapeDtypeStruct((NC, NPAD, LAT // 2), jnp.float32),
        jax.ShapeDtypeStruct((NC, NPAD, LAT // 2), jnp.float32),
        jax.ShapeDtypeStruct((NC, NPAD), jnp.float32),
    ),
    mesh=plsc.VectorSubcoreMesh(core_axis_name="c", subcore_axis_name="s"),
    compiler_params=pltpu.CompilerParams(needs_layout_passes=False,
                                         use_tc_tiling_on_sc=False),
    scratch_types=[
        pltpu.VMEM((NPAD,), jnp.float32),
        pltpu.VMEM((NPAD,), jnp.float32),
        pltpu.VMEM((16,), jnp.float32),
        pltpu.VMEM((EW,), jnp.float32),
        pltpu.VMEM((EW,), jnp.int32),
        pltpu.VMEM((EW,), jnp.int32),
        pltpu.VMEM((NB, 128), jnp.int32),
        pltpu.VMEM((EW,), jnp.float32),
        pltpu.VMEM((128, LAT // 2), jnp.float32),
        pltpu.VMEM((640,), jnp.float32),
        pltpu.VMEM_SHARED((NPAD, LAT // 2), jnp.float32),
        pltpu.VMEM_SHARED((NPAD,), jnp.float32),
        pltpu.SemaphoreType.DMA,
    ],
)(_sc_body)


FBLK = 1264


def _mlp_body(plo_ref, phi_ref, den_ref, w1_ref, b1_ref, w2_ref, b2_ref,
              av_ref, v_ref, minv_ref, m1_ref):
    den = den_ref[0] + den_ref[1] + 1e-9
    o = jnp.concatenate([plo_ref[0] + plo_ref[1], phi_ref[0] + phi_ref[1]],
                        axis=1) / den
    h = jnp.where(o > 0, o, (jnp.exp(o) - 1.0))
    t1 = jnp.maximum(
        jnp.dot(h, w1_ref[...], precision=_HI,
                preferred_element_type=jnp.float32) + b1_ref[...], 0.0)
    v = jnp.dot(t1, w2_ref[...], precision=_HI,
                preferred_element_type=jnp.float32) + b2_ref[...]
    v_ref[...] = v
    i = pl.program_id(0)
    rows = lax.broadcasted_iota(jnp.int32, (FBLK, 1), 0) + i * FBLK
    mask = rows < NN
    av = av_ref[...]
    bmin = jnp.full((1, 1), jnp.min(jnp.where(mask, v, jnp.inf)))
    bm1 = jnp.full((1, 1),
                   jnp.max(jnp.where(mask & (av > 0), v, -jnp.inf)))
    pmin = jnp.where(i == 0, jnp.full((1, 1), jnp.inf), minv_ref[...])
    pm1 = jnp.where(i == 0, jnp.full((1, 1), -jnp.inf), m1_ref[...])
    minv_ref[...] = jnp.minimum(pmin, bmin)
    m1_ref[...] = jnp.maximum(pm1, bm1)


def _mlp_call(plo, phi, den, W1, b1, W2, b2, av):
    return pl.pallas_call(
        _mlp_body,
        grid=(NPAD // FBLK,),
        in_specs=[
            pl.BlockSpec((NC, FBLK, LAT // 2), lambda i: (0, i, 0)),
            pl.BlockSpec((NC, FBLK, LAT // 2), lambda i: (0, i, 0)),
            pl.BlockSpec((NC, FBLK, 1), lambda i: (0, i, 0)),
            pl.BlockSpec((LAT, HID), lambda i: (0, 0)),
            pl.BlockSpec((1, HID), lambda i: (0, 0)),
            pl.BlockSpec((HID, 1), lambda i: (0, 0)),
            pl.BlockSpec((1, 1), lambda i: (0, 0)),
            pl.BlockSpec((FBLK, 1), lambda i: (i, 0)),
        ],
        out_specs=[
            pl.BlockSpec((FBLK, 1), lambda i: (i, 0)),
            pl.BlockSpec((1, 1), lambda i: (0, 0)),
            pl.BlockSpec((1, 1), lambda i: (0, 0)),
        ],
        out_shape=[
            jax.ShapeDtypeStruct((NPAD, 1), jnp.float32),
            jax.ShapeDtypeStruct((1, 1), jnp.float32),
            jax.ShapeDtypeStruct((1, 1), jnp.float32),
        ],
    )(plo, phi, den, W1, b1, W2, b2, av)


def _soft_body(v_ref, av_ref, cz_ref, num_ref, ssum_ref, nsum_ref):
    i = pl.program_id(0)
    rows = lax.broadcasted_iota(jnp.int32, (FBLK, 1), 0) + i * FBLK
    mask = rows < NN
    av = av_ref[...]
    z2 = (v_ref[...] + cz_ref[...]) * av
    sv = jnp.where(mask, jnp.exp(z2), 0.0)
    num = sv * av
    num_ref[...] = num
    ps = jnp.where(i == 0, jnp.zeros((1, 1)), ssum_ref[...])
    pn = jnp.where(i == 0, jnp.zeros((1, 1)), nsum_ref[...])
    ssum_ref[...] = ps + jnp.full((1, 1), jnp.sum(sv))
    nsum_ref[...] = pn + jnp.full((1, 1), jnp.sum(num))


def _soft_call(v, av, cz):
    return pl.pallas_call(
        _soft_body,
        grid=(NPAD // FBLK,),
        in_specs=[
            pl.BlockSpec((FBLK, 1), lambda i: (i, 0)),
            pl.BlockSpec((FBLK, 1), lambda i: (i, 0)),
            pl.BlockSpec((1, 1), lambda i: (0, 0)),
        ],
        out_specs=[
            pl.BlockSpec((FBLK, 1), lambda i: (i, 0)),
            pl.BlockSpec((1, 1), lambda i: (0, 0)),
            pl.BlockSpec((1, 1), lambda i: (0, 0)),
        ],
        out_shape=[
            jax.ShapeDtypeStruct((NPAD, 1), jnp.float32),
            jax.ShapeDtypeStruct((1, 1), jnp.float32),
            jax.ShapeDtypeStruct((1, 1), jnp.float32),
        ],
    )(v, av, cz)


def _scale_body(num_ref, inv_ref, out_ref):
    out_ref[...] = num_ref[...] * inv_ref[...]


def _scale_call(num, inv):
    return pl.pallas_call(
        _scale_body,
        grid=(NPAD // FBLK,),
        in_specs=[
            pl.BlockSpec((FBLK, 1), lambda i: (i, 0)),
            pl.BlockSpec((1, 1), lambda i: (0, 0)),
        ],
        out_specs=pl.BlockSpec((FBLK, 1), lambda i: (i, 0)),
        out_shape=jax.ShapeDtypeStruct((NPAD, 1), jnp.float32),
    )(num, inv)


def kernel(x, edge_index, edge_attr, available, params):
    pad_e = EPAD - EE
    src_p = jnp.concatenate(
        [edge_index[0], jnp.full((pad_e,), NN, jnp.int32)])
    dst_p = jnp.concatenate(
        [edge_index[1], jnp.full((pad_e,), NN, jnp.int32)])
    dst2d = jnp.concatenate(
        [edge_index[1].reshape(EE // 128, 128),
         jnp.full((pad_e // 128, 128), NN, jnp.int32)], axis=0)
    x_p = jnp.pad(x, ((0, NPAD - NN), (0, 0)))
    e_cur = jnp.pad(edge_attr, ((0, pad_e), (0, 0)))
    av_p = jnp.pad(available, (0, NPAD - NN)).reshape(NPAD, 1)

    plo = phi = den = None
    for l in range(5):
        a = params[f'a{l}']
        a01 = jnp.stack([a[:LAT], a[LAT:2 * LAT]], axis=1)
        a2 = a[2 * LAT:].reshape(LAT, 1)
        en, se, semax = _edge_call(e_cur, params[f'We{l}'], a2)
        if l == 0:
            hslo, hshi, s0, s1, s0max = _node0_call(x_p, params[f'Wn{l}'],
                                                    a01)
        else:
            hslo, hshi, s0, s1, s0max = _nodec_call(plo, phi, den,
                                                    params[f'Wn{l}'], a01)
        m16 = jnp.full((16,), s0max[0, 0] + semax[0, 0], jnp.float32)
        plo, phi, den2 = _sc_call(hslo, hshi, s0.reshape(NPAD),
                                  s1.reshape(NPAD), m16, se.reshape(EPAD),
                                  src_p, dst_p, dst2d)
        den = den2.reshape(NC, NPAD, 1)
        e_cur = en

    v, minv, m1 = _mlp_call(plo, phi, den, params['W1'],
                            params['b1'].reshape(1, HID), params['W2'],
                            params['b2'].reshape(1, 1), av_p)
    c = jnp.abs(minv[0, 0])
    zm = jnp.maximum(0.0, m1[0, 0] + c)
    cz = jnp.full((1, 1), c - zm, jnp.float32)
    num, ssum, nsum = _soft_call(v, av_p, cz)
    inv = (1.0 / (nsum + 1e-13 * ssum)).reshape(1, 1)
    out = _scale_call(num, inv)
    return out.reshape(NPAD)[:NN]


# speedup vs baseline: 8.5875x; 1.0288x over previous
import functools

import jax
import jax.numpy as jnp
from jax import lax
from jax.experimental import pallas as pl
from jax.experimental.pallas import tpu as pltpu
from jax.experimental.pallas import tpu_sc as plsc

NN = 10000
EE = 320000
LAT = 64
HID = 128
NPAD = 10112
EPAD = 327680
NC, NS = 2, 16
NW = NC * NS
EW = EPAD // NW
NB = EW // 128
NROWS = NPAD // NS

_HI = lax.Precision.HIGHEST


def _edge_body(e_ref, we_ref, a2_ref, en_ref, se_ref, semax_ref):
    et = jnp.dot(e_ref[...], we_ref[...], precision=_HI,
                 preferred_element_type=jnp.float32)
    en_ref[...] = jnp.where(et > 0, et, (jnp.exp(et) - 1.0))
    sev = jnp.dot(et, a2_ref[...], precision=_HI,
                  preferred_element_type=jnp.float32)
    se_ref[...] = sev
    i = pl.program_id(0)
    prev = jnp.where(i == 0, jnp.full((1, 1), -jnp.inf), semax_ref[...])
    semax_ref[...] = jnp.maximum(prev, jnp.full((1, 1), jnp.max(sev)))


def _edge_call(e, We, a2):
    rb = 2048
    de = e.shape[1]
    grid = EPAD // rb
    return pl.pallas_call(
        _edge_body,
        grid=(grid,),
        in_specs=[
            pl.BlockSpec((rb, de), lambda i: (i, 0)),
            pl.BlockSpec((de, LAT), lambda i: (0, 0)),
            pl.BlockSpec((LAT, 1), lambda i: (0, 0)),
        ],
        out_specs=[
            pl.BlockSpec((rb, LAT), lambda i: (i, 0)),
            pl.BlockSpec((rb, 1), lambda i: (i, 0)),
            pl.BlockSpec((1, 1), lambda i: (0, 0)),
        ],
        out_shape=[
            jax.ShapeDtypeStruct((EPAD, LAT), jnp.float32),
            jax.ShapeDtypeStruct((EPAD, 1), jnp.float32),
            jax.ShapeDtypeStruct((1, 1), jnp.float32),
        ],
    )(e, We, a2)


NBLK = 1264


def _node_tail(h, wn_ref, a01_ref, hslo_ref, hshi_ref, s0_ref, s1_ref,
               s0max_ref):
    hs = jnp.dot(h, wn_ref[...], precision=_HI,
                 preferred_element_type=jnp.float32)
    hslo_ref[...] = hs[:, :LAT // 2]
    hshi_ref[...] = hs[:, LAT // 2:]
    s = jnp.dot(hs, a01_ref[...], precision=_HI,
                preferred_element_type=jnp.float32)
    s0_ref[...] = s[:, 0:1]
    s1_ref[...] = s[:, 1:2]
    i = pl.program_id(0)
    prev = jnp.where(i == 0, jnp.full((1, 1), -jnp.inf), s0max_ref[...])
    s0max_ref[...] = jnp.maximum(prev, jnp.full((1, 1), jnp.max(s[:, 0])))


def _node0_body(x_ref, wn_ref, a01_ref, hslo_ref, hshi_ref, s0_ref, s1_ref,
                s0max_ref):
    _node_tail(x_ref[...], wn_ref, a01_ref, hslo_ref, hshi_ref, s0_ref,
               s1_ref, s0max_ref)


def _nodec_body(plo_ref, phi_ref, den_ref, wn_ref, a01_ref, hslo_ref,
                hshi_ref, s0_ref, s1_ref, s0max_ref):
    den = den_ref[0] + den_ref[1] + 1e-9
    o = jnp.concatenate([plo_ref[0] + plo_ref[1], phi_ref[0] + phi_ref[1]],
                        axis=1) / den
    h = jnp.where(o > 0, o, (jnp.exp(o) - 1.0))
    _node_tail(h, wn_ref, a01_ref, hslo_ref, hshi_ref, s0_ref, s1_ref,
               s0max_ref)


_NODE_OUT = [
    jax.ShapeDtypeStruct((NPAD, LAT // 2), jnp.float32),
    jax.ShapeDtypeStruct((NPAD, LAT // 2), jnp.float32),
    jax.ShapeDtypeStruct((NPAD, 1), jnp.float32),
    jax.ShapeDtypeStruct((NPAD, 1), jnp.float32),
    jax.ShapeDtypeStruct((1, 1), jnp.float32),
]

_NODE_OUT_SPECS = [
    pl.BlockSpec((NBLK, LAT // 2), lambda i: (i, 0)),
    pl.BlockSpec((NBLK, LAT // 2), lambda i: (i, 0)),
    pl.BlockSpec((NBLK, 1), lambda i: (i, 0)),
    pl.BlockSpec((NBLK, 1), lambda i: (i, 0)),
    pl.BlockSpec((1, 1), lambda i: (0, 0)),
]


def _node0_call(x, Wn, a01):
    dn = x.shape[1]
    return pl.pallas_call(
        _node0_body,
        grid=(NPAD // NBLK,),
        in_specs=[
            pl.BlockSpec((NBLK, dn), lambda i: (i, 0)),
            pl.BlockSpec((dn, LAT), lambda i: (0, 0)),
            pl.BlockSpec((LAT, 2), lambda i: (0, 0)),
        ],
        out_specs=_NODE_OUT_SPECS,
        out_shape=_NODE_OUT,
    )(x, Wn, a01)


def _nodec_call(plo, phi, den, Wn, a01):
    return pl.pallas_call(
        _nodec_body,
        grid=(NPAD // NBLK,),
        in_specs=[
            pl.BlockSpec((NC, NBLK, LAT // 2), lambda i: (0, i, 0)),
            pl.BlockSpec((NC, NBLK, LAT // 2), lambda i: (0, i, 0)),
            pl.BlockSpec((NC, NBLK, 1), lambda i: (0, i, 0)),
            pl.BlockSpec((LAT, LAT), lambda i: (0, 0)),
            pl.BlockSpec((LAT, 2), lambda i: (0, 0)),
        ],
        out_specs=_NODE_OUT_SPECS,
        out_shape=_NODE_OUT,
    )(plo, phi, den, Wn, a01)


def _sc_body(hslo_hbm, hshi_hbm, s0_hbm, s1_hbm, m_hbm, se_hbm, src_hbm,
             dst_hbm, dst2d_hbm, outlo_hbm, outhi_hbm, den_hbm, s0_v, s1_v,
             m_v, se_v, src_v, dst_v, dst_m, ex_v, r0, r1, r2, r3, zrow_v,
             zden_v, shared_out, shared_den, gs0, gs1, gs2, gs3, ss0, ss1,
             ss2, ss3, dsem):
    cid = lax.axis_index("c")
    sid = lax.axis_index("s")
    wid = sid * NC + cid
    ebase = wid * EW
    rbase = wid * NB
    HLAT = LAT // 2
    rows = (r0, r1, r2, r3)
    gsems = (gs0, gs1, gs2, gs3)
    ssems = (ss0, ss1, ss2, ss3)

    pltpu.sync_copy(s0_hbm, s0_v)
    pltpu.sync_copy(s1_hbm, s1_v)
    pltpu.sync_copy(m_hbm, m_v)
    pltpu.sync_copy(se_hbm.at[pl.ds(ebase, EW)], se_v)
    pltpu.sync_copy(src_hbm.at[pl.ds(ebase, EW)], src_v)
    pltpu.sync_copy(dst_hbm.at[pl.ds(ebase, EW)], dst_v)
    pltpu.sync_copy(dst2d_hbm.at[pl.ds(rbase, NB)], dst_m)

    zeros16 = jnp.zeros((16,), jnp.float32)

    def zrow(r, _):
        for c in range(HLAT // 16):
            zrow_v[r, pl.ds(c * 16, 16)] = zeros16
        return 0

    lax.fori_loop(0, 128, zrow, 0)

    def zden(r, _):
        zden_v[pl.ds(r * 16, 16)] = zeros16
        return 0

    lax.fori_loop(0, 640 // 16, zden, 0)

    def _zero_out_slice():
        for r in range(NROWS // 128):
            pltpu.sync_copy(
                zrow_v, shared_out.at[pl.ds(sid * NROWS + r * 128, 128)])
        _rem = NROWS % 128
        if _rem:
            pltpu.sync_copy(
                zrow_v.at[pl.ds(0, _rem)],
                shared_out.at[pl.ds(sid * NROWS + (NROWS // 128) * 128,
                                    _rem)])

    def _den_desc(j):
        return pltpu.make_async_copy(ex_v.at[pl.ds(j * 128, 128)],
                                     shared_den.at[dst_m.at[j]], dsem)

    def _ex_phase():
        def exbatch(j, carry):
            for i in range(8):
                b = j * 128 + i * 16
                s16 = src_v[pl.ds(b, 16)]
                d16 = dst_v[pl.ds(b, 16)]
                g0 = plsc.load_gather(s0_v, [s16])
                g1 = plsc.load_gather(s1_v, [d16])
                lg = g0 + g1 + se_v[pl.ds(b, 16)]
                lg = jnp.maximum(lg, lg * 0.2)
                pre = m_v[...] + g1
                gt = jnp.maximum(pre, pre * 0.2)
                ex_v[pl.ds(b, 16)] = jnp.exp(lg - gt)

            @pl.when(j >= 1)
            def _():
                _den_desc(j - 1).wait()

            pltpu.async_copy(ex_v.at[pl.ds(j * 128, 128)],
                             shared_den.at[dst_m.at[j]], dsem, add=True)
            return carry

        lax.fori_loop(0, NB, exbatch, 0)
        _den_desc(NB - 1).wait()

    def _g_issue(j, b, hs_hbm):
        pltpu.async_copy(hs_hbm.at[src_v.at[pl.ds(j * 128, 128)]], rows[b],
                         gsems[b])

    def _g_wait(j, b, hs_hbm):
        pltpu.make_async_copy(hs_hbm.at[src_v.at[pl.ds(j * 128, 128)]],
                              rows[b], gsems[b]).wait()

    def _s_issue(j, b):
        pltpu.async_copy(rows[b], shared_out.at[dst_m.at[j]], ssems[b],
                         add=True)

    def _s_wait(j, b):
        pltpu.make_async_copy(rows[b], shared_out.at[dst_m.at[j]],
                              ssems[b]).wait()

    def _pipe(hs_hbm):
        _g_issue(0, 0, hs_hbm)
        _g_issue(1, 1, hs_hbm)

        def outer(i, carry):
            for b4 in range(4):
                j = i * 4 + b4
                _g_wait(j, b4, hs_hbm)
                rv = rows[b4]

                def scale16(k, c2):
                    ex16 = ex_v[pl.ds(j * 128 + k * 16, 16)]
                    for e in range(16):
                        a = ex16[e]
                        r = k * 16 + e
                        for c in range(HLAT // 16):
                            sl = pl.ds(c * 16, 16)
                            rv[r, sl] = rv[r, sl] * a
                    return c2

                lax.fori_loop(0, 8, scale16, 0)
                _s_issue(j, b4)
                b2 = (b4 + 2) % 4

                @pl.when(j + 2 < NB)
                def _():
                    @pl.when(j >= 2)
                    def __():
                        _s_wait(j - 2, b2)

                    _g_issue(j + 2, b2, hs_hbm)
            return carry

        lax.fori_loop(0, NB // 4, outer, 0)
        for jj in range(NB - 4, NB):
            _s_wait(jj, jj % 4)

    for half in range(2):
        hs_hbm = hslo_hbm if half == 0 else hshi_hbm
        out_hbm = outlo_hbm if half == 0 else outhi_hbm

        _zero_out_slice()
        if half == 0:
            pltpu.sync_copy(zden_v.at[pl.ds(0, NROWS)],
                            shared_den.at[pl.ds(sid * NROWS, NROWS)])
        plsc.subcore_barrier()

        if half == 0:
            _ex_phase()
        _pipe(hs_hbm)
        plsc.subcore_barrier()

        pltpu.sync_copy(shared_out.at[pl.ds(sid * NROWS, NROWS)],
                        out_hbm.at[cid, pl.ds(sid * NROWS, NROWS)])
        if half == 0:
            pltpu.sync_copy(shared_den.at[pl.ds(sid * NROWS, NROWS)],
                            den_hbm.at[cid, pl.ds(sid * NROWS, NROWS)])


_sc_call = functools.partial(
    pl.kernel,
    out_type=(
        jax.ShapeDtypeStruct((NC, NPAD, LAT // 2), jnp.float32),
        jax.ShapeDtypeStruct((NC, NPAD, LAT // 2), jnp.float32),
        jax.ShapeDtypeStruct((NC, NPAD), jnp.float32),
    ),
    mesh=plsc.VectorSubcoreMesh(core_axis_name="c", subcore_axis_name="s"),
    compiler_params=pltpu.CompilerParams(needs_layout_passes=False,
                                         use_tc_tiling_on_sc=False),
    scratch_types=[
        pltpu.VMEM((NPAD,), jnp.float32),
        pltpu.VMEM((NPAD,), jnp.float32),
        pltpu.VMEM((16,), jnp.float32),
        pltpu.VMEM((EW,), jnp.float32),
        pltpu.VMEM((EW,), jnp.int32),
        pltpu.VMEM((EW,), jnp.int32),
        pltpu.VMEM((NB, 128), jnp.int32),
        pltpu.VMEM((EW,), jnp.float32),
        pltpu.VMEM((128, LAT // 2), jnp.float32),
        pltpu.VMEM((128, LAT // 2), jnp.float32),
        pltpu.VMEM((128, LAT // 2), jnp.float32),
        pltpu.VMEM((128, LAT // 2), jnp.float32),
        pltpu.VMEM((128, LAT // 2), jnp.float32),
        pltpu.VMEM((640,), jnp.float32),
        pltpu.VMEM_SHARED((NPAD, LAT // 2), jnp.float32),
        pltpu.VMEM_SHARED((NPAD,), jnp.float32),
        pltpu.SemaphoreType.DMA,
        pltpu.SemaphoreType.DMA,
        pltpu.SemaphoreType.DMA,
        pltpu.SemaphoreType.DMA,
        pltpu.SemaphoreType.DMA,
        pltpu.SemaphoreType.DMA,
        pltpu.SemaphoreType.DMA,
        pltpu.SemaphoreType.DMA,
        pltpu.SemaphoreType.DMA,
    ],
)(_sc_body)


FBLK = 1264


def _mlp_body(plo_ref, phi_ref, den_ref, w1_ref, b1_ref, w2_ref, b2_ref,
              av_ref, v_ref, minv_ref, m1_ref):
    den = den_ref[0] + den_ref[1] + 1e-9
    o = jnp.concatenate([plo_ref[0] + plo_ref[1], phi_ref[0] + phi_ref[1]],
                        axis=1) / den
    h = jnp.where(o > 0, o, (jnp.exp(o) - 1.0))
    t1 = jnp.maximum(
        jnp.dot(h, w1_ref[...], precision=_HI,
                preferred_element_type=jnp.float32) + b1_ref[...], 0.0)
    v = jnp.dot(t1, w2_ref[...], precision=_HI,
                preferred_element_type=jnp.float32) + b2_ref[...]
    v_ref[...] = v
    i = pl.program_id(0)
    rows = lax.broadcasted_iota(jnp.int32, (FBLK, 1), 0) + i * FBLK
    mask = rows < NN
    av = av_ref[...]
    bmin = jnp.full((1, 1), jnp.min(jnp.where(mask, v, jnp.inf)))
    bm1 = jnp.full((1, 1),
                   jnp.max(jnp.where(mask & (av > 0), v, -jnp.inf)))
    pmin = jnp.where(i == 0, jnp.full((1, 1), jnp.inf), minv_ref[...])
    pm1 = jnp.where(i == 0, jnp.full((1, 1), -jnp.inf), m1_ref[...])
    minv_ref[...] = jnp.minimum(pmin, bmin)
    m1_ref[...] = jnp.maximum(pm1, bm1)


def _mlp_call(plo, phi, den, W1, b1, W2, b2, av):
    return pl.pallas_call(
        _mlp_body,
        grid=(NPAD // FBLK,),
        in_specs=[
            pl.BlockSpec((NC, FBLK, LAT // 2), lambda i: (0, i, 0)),
            pl.BlockSpec((NC, FBLK, LAT // 2), lambda i: (0, i, 0)),
            pl.BlockSpec((NC, FBLK, 1), lambda i: (0, i, 0)),
            pl.BlockSpec((LAT, HID), lambda i: (0, 0)),
            pl.BlockSpec((1, HID), lambda i: (0, 0)),
            pl.BlockSpec((HID, 1), lambda i: (0, 0)),
            pl.BlockSpec((1, 1), lambda i: (0, 0)),
            pl.BlockSpec((FBLK, 1), lambda i: (i, 0)),
        ],
        out_specs=[
            pl.BlockSpec((FBLK, 1), lambda i: (i, 0)),
            pl.BlockSpec((1, 1), lambda i: (0, 0)),
            pl.BlockSpec((1, 1), lambda i: (0, 0)),
        ],
        out_shape=[
            jax.ShapeDtypeStruct((NPAD, 1), jnp.float32),
            jax.ShapeDtypeStruct((1, 1), jnp.float32),
            jax.ShapeDtypeStruct((1, 1), jnp.float32),
        ],
    )(plo, phi, den, W1, b1, W2, b2, av)


def _soft_body(v_ref, av_ref, cz_ref, num_ref, ssum_ref, nsum_ref):
    i = pl.program_id(0)
    rows = lax.broadcasted_iota(jnp.int32, (FBLK, 1), 0) + i * FBLK
    mask = rows < NN
    av = av_ref[...]
    z2 = (v_ref[...] + cz_ref[...]) * av
    sv = jnp.where(mask, jnp.exp(z2), 0.0)
    num = sv * av
    num_ref[...] = num
    ps = jnp.where(i == 0, jnp.zeros((1, 1)), ssum_ref[...])
    pn = jnp.where(i == 0, jnp.zeros((1, 1)), nsum_ref[...])
    ssum_ref[...] = ps + jnp.full((1, 1), jnp.sum(sv))
    nsum_ref[...] = pn + jnp.full((1, 1), jnp.sum(num))


def _soft_call(v, av, cz):
    return pl.pallas_call(
        _soft_body,
        grid=(NPAD // FBLK,),
        in_specs=[
            pl.BlockSpec((FBLK, 1), lambda i: (i, 0)),
            pl.BlockSpec((FBLK, 1), lambda i: (i, 0)),
            pl.BlockSpec((1, 1), lambda i: (0, 0)),
        ],
        out_specs=[
            pl.BlockSpec((FBLK, 1), lambda i: (i, 0)),
            pl.BlockSpec((1, 1), lambda i: (0, 0)),
            pl.BlockSpec((1, 1), lambda i: (0, 0)),
        ],
        out_shape=[
            jax.ShapeDtypeStruct((NPAD, 1), jnp.float32),
            jax.ShapeDtypeStruct((1, 1), jnp.float32),
            jax.ShapeDtypeStruct((1, 1), jnp.float32),
        ],
    )(v, av, cz)


def _scale_body(num_ref, inv_ref, out_ref):
    out_ref[...] = num_ref[...] * inv_ref[...]


def _scale_call(num, inv):
    return pl.pallas_call(
        _scale_body,
        grid=(NPAD // FBLK,),
        in_specs=[
            pl.BlockSpec((FBLK, 1), lambda i: (i, 0)),
            pl.BlockSpec((1, 1), lambda i: (0, 0)),
        ],
        out_specs=pl.BlockSpec((FBLK, 1), lambda i: (i, 0)),
        out_shape=jax.ShapeDtypeStruct((NPAD, 1), jnp.float32),
    )(num, inv)


def kernel(x, edge_index, edge_attr, available, params):
    pad_e = EPAD - EE
    src_p = jnp.concatenate(
        [edge_index[0], jnp.full((pad_e,), NN, jnp.int32)])
    dst_p = jnp.concatenate(
        [edge_index[1], jnp.full((pad_e,), NN, jnp.int32)])
    dst2d = jnp.concatenate(
        [edge_index[1].reshape(EE // 128, 128),
         jnp.full((pad_e // 128, 128), NN, jnp.int32)], axis=0)
    x_p = jnp.pad(x, ((0, NPAD - NN), (0, 0)))
    e_cur = jnp.pad(edge_attr, ((0, pad_e), (0, 0)))
    av_p = jnp.pad(available, (0, NPAD - NN)).reshape(NPAD, 1)

    plo = phi = den = None
    for l in range(5):
        a = params[f'a{l}']
        a01 = jnp.stack([a[:LAT], a[LAT:2 * LAT]], axis=1)
        a2 = a[2 * LAT:].reshape(LAT, 1)
        en, se, semax = _edge_call(e_cur, params[f'We{l}'], a2)
        if l == 0:
            hslo, hshi, s0, s1, s0max = _node0_call(x_p, params[f'Wn{l}'],
                                                    a01)
        else:
            hslo, hshi, s0, s1, s0max = _nodec_call(plo, phi, den,
                                                    params[f'Wn{l}'], a01)
        m16 = jnp.full((16,), s0max[0, 0] + semax[0, 0], jnp.float32)
        plo, phi, den2 = _sc_call(hslo, hshi, s0.reshape(NPAD),
                                  s1.reshape(NPAD), m16, se.reshape(EPAD),
                                  src_p, dst_p, dst2d)
        den = den2.reshape(NC, NPAD, 1)
        e_cur = en

    v, minv, m1 = _mlp_call(plo, phi, den, params['W1'],
                            params['b1'].reshape(1, HID), params['W2'],
                            params['b2'].reshape(1, 1), av_p)
    c = jnp.abs(minv[0, 0])
    zm = jnp.maximum(0.0, m1[0, 0] + c)
    cz = jnp.full((1, 1), c - zm, jnp.float32)
    num, ssum, nsum = _soft_call(v, av_p, cz)
    inv = (1.0 / (nsum + 1e-13 * ssum)).reshape(1, 1)
    out = _scale_call(num, inv)
    return out.reshape(NPAD)[:NN]


# speedup vs baseline: 15.5689x; 1.8130x over previous
import functools

import jax
import jax.numpy as jnp
from jax import lax
from jax.experimental import pallas as pl
from jax.experimental.pallas import tpu as pltpu
from jax.experimental.pallas import tpu_sc as plsc

NN = 10000
EE = 320000
LAT = 64
HID = 128
NPAD = 10112
EPAD = 327680
NC, NS = 2, 16
NW = NC * NS
EW = EPAD // NW
NB = EW // 128
NROWS = NPAD // NS

_HI = lax.Precision.HIGHEST


def _edge_body(eT_ref, weT_ref, a2r_ref, enT_ref, seT_ref, semax_ref):
    etT = jnp.dot(weT_ref[...], eT_ref[...], precision=_HI,
                  preferred_element_type=jnp.float32)
    enT_ref[...] = jnp.where(etT > 0, etT, (jnp.exp(etT) - 1.0))
    sev = jnp.dot(a2r_ref[...], etT, precision=_HI,
                  preferred_element_type=jnp.float32)
    seT_ref[...] = sev
    i = pl.program_id(0)
    prev = jnp.where(i == 0, jnp.full((1, 1), -jnp.inf), semax_ref[...])
    semax_ref[...] = jnp.maximum(prev, jnp.full((1, 1), jnp.max(sev)))


def _edge_call(eT, WeT, a2r):
    rb = 4096
    de = eT.shape[0]
    return pl.pallas_call(
        _edge_body,
        grid=(EPAD // rb,),
        in_specs=[
            pl.BlockSpec((de, rb), lambda i: (0, i)),
            pl.BlockSpec((LAT, de), lambda i: (0, 0)),
            pl.BlockSpec((1, LAT), lambda i: (0, 0)),
        ],
        out_specs=[
            pl.BlockSpec((LAT, rb), lambda i: (0, i)),
            pl.BlockSpec((1, rb), lambda i: (0, i)),
            pl.BlockSpec((1, 1), lambda i: (0, 0)),
        ],
        out_shape=[
            jax.ShapeDtypeStruct((LAT, EPAD), jnp.float32),
            jax.ShapeDtypeStruct((1, EPAD), jnp.float32),
            jax.ShapeDtypeStruct((1, 1), jnp.float32),
        ],
    )(eT, WeT, a2r)


NBLK = 1264


def _node_tail(h, wn_ref, a01_ref, hslo_ref, hshi_ref, s0_ref, s1_ref,
               s0max_ref):
    hs = jnp.dot(h, wn_ref[...], precision=_HI,
                 preferred_element_type=jnp.float32)
    hslo_ref[...] = hs[:, :LAT // 2]
    hshi_ref[...] = hs[:, LAT // 2:]
    s = jnp.dot(hs, a01_ref[...], precision=_HI,
                preferred_element_type=jnp.float32)
    s0_ref[...] = s[:, 0:1]
    s1_ref[...] = s[:, 1:2]
    i = pl.program_id(0)
    prev = jnp.where(i == 0, jnp.full((1, 1), -jnp.inf), s0max_ref[...])
    s0max_ref[...] = jnp.maximum(prev, jnp.full((1, 1), jnp.max(s[:, 0])))


def _node0_body(x_ref, wn_ref, a01_ref, hslo_ref, hshi_ref, s0_ref, s1_ref,
                s0max_ref):
    _node_tail(x_ref[...], wn_ref, a01_ref, hslo_ref, hshi_ref, s0_ref,
               s1_ref, s0max_ref)


def _nodec_body(plo_ref, phi_ref, den_ref, wn_ref, a01_ref, hslo_ref,
                hshi_ref, s0_ref, s1_ref, s0max_ref):
    den = den_ref[0] + den_ref[1] + 1e-9
    o = jnp.concatenate([plo_ref[0] + plo_ref[1], phi_ref[0] + phi_ref[1]],
                        axis=1) / den
    h = jnp.where(o > 0, o, (jnp.exp(o) - 1.0))
    _node_tail(h, wn_ref, a01_ref, hslo_ref, hshi_ref, s0_ref, s1_ref,
               s0max_ref)


_NODE_OUT = [
    jax.ShapeDtypeStruct((NPAD, LAT // 2), jnp.float32),
    jax.ShapeDtypeStruct((NPAD, LAT // 2), jnp.float32),
    jax.ShapeDtypeStruct((NPAD, 1), jnp.float32),
    jax.ShapeDtypeStruct((NPAD, 1), jnp.float32),
    jax.ShapeDtypeStruct((1, 1), jnp.float32),
]

_NODE_OUT_SPECS = [
    pl.BlockSpec((NBLK, LAT // 2), lambda i: (i, 0)),
    pl.BlockSpec((NBLK, LAT // 2), lambda i: (i, 0)),
    pl.BlockSpec((NBLK, 1), lambda i: (i, 0)),
    pl.BlockSpec((NBLK, 1), lambda i: (i, 0)),
    pl.BlockSpec((1, 1), lambda i: (0, 0)),
]


def _node0_call(x, Wn, a01):
    dn = x.shape[1]
    return pl.pallas_call(
        _node0_body,
        grid=(NPAD // NBLK,),
        in_specs=[
            pl.BlockSpec((NBLK, dn), lambda i: (i, 0)),
            pl.BlockSpec((dn, LAT), lambda i: (0, 0)),
            pl.BlockSpec((LAT, 2), lambda i: (0, 0)),
        ],
        out_specs=_NODE_OUT_SPECS,
        out_shape=_NODE_OUT,
    )(x, Wn, a01)


def _nodec_call(plo, phi, den, Wn, a01):
    return pl.pallas_call(
        _nodec_body,
        grid=(NPAD // NBLK,),
        in_specs=[
            pl.BlockSpec((NC, NBLK, LAT // 2), lambda i: (0, i, 0)),
            pl.BlockSpec((NC, NBLK, LAT // 2), lambda i: (0, i, 0)),
            pl.BlockSpec((NC, NBLK, 1), lambda i: (0, i, 0)),
            pl.BlockSpec((LAT, LAT), lambda i: (0, 0)),
            pl.BlockSpec((LAT, 2), lambda i: (0, 0)),
        ],
        out_specs=_NODE_OUT_SPECS,
        out_shape=_NODE_OUT,
    )(plo, phi, den, Wn, a01)


def _sc_body(hslo_hbm, hshi_hbm, s0_hbm, s1_hbm, m_hbm, se_hbm, src_hbm,
             dst_hbm, dst2d_hbm, outlo_hbm, outhi_hbm, den_hbm, s0_v, s1_v,
             m_v, se_v, src_v, dst_v, dst_m, ex_v, r0, r1, r2, r3, zrow_v,
             zden_v, shared_out, shared_den, gs0, gs1, gs2, gs3, ss0, ss1,
             ss2, ss3, dsem):
    cid = lax.axis_index("c")
    sid = lax.axis_index("s")
    wid = sid * NC + cid
    ebase = wid * EW
    rbase = wid * NB
    HLAT = LAT // 2
    rows = (r0, r1, r2, r3)
    gsems = (gs0, gs1, gs2, gs3)
    ssems = (ss0, ss1, ss2, ss3)

    pltpu.sync_copy(s0_hbm, s0_v)
    pltpu.sync_copy(s1_hbm, s1_v)
    pltpu.sync_copy(m_hbm, m_v)
    pltpu.sync_copy(se_hbm.at[pl.ds(ebase, EW)], se_v)
    pltpu.sync_copy(src_hbm.at[pl.ds(ebase, EW)], src_v)
    pltpu.sync_copy(dst_hbm.at[pl.ds(ebase, EW)], dst_v)
    pltpu.sync_copy(dst2d_hbm.at[pl.ds(rbase, NB)], dst_m)

    zeros16 = jnp.zeros((16,), jnp.float32)

    def zrow(r, _):
        for c in range(HLAT // 16):
            zrow_v[r, pl.ds(c * 16, 16)] = zeros16
        return 0

    lax.fori_loop(0, 128, zrow, 0)

    def zden(r, _):
        zden_v[pl.ds(r * 16, 16)] = zeros16
        return 0

    lax.fori_loop(0, 640 // 16, zden, 0)

    def _zero_out_slice():
        for r in range(NROWS // 128):
            pltpu.sync_copy(
                zrow_v, shared_out.at[pl.ds(sid * NROWS + r * 128, 128)])
        _rem = NROWS % 128
        if _rem:
            pltpu.sync_copy(
                zrow_v.at[pl.ds(0, _rem)],
                shared_out.at[pl.ds(sid * NROWS + (NROWS // 128) * 128,
                                    _rem)])

    def _den_desc(j):
        return pltpu.make_async_copy(ex_v.at[pl.ds(j * 128, 128)],
                                     shared_den.at[dst_m.at[j]], dsem)

    def _ex_phase():
        def exbatch(j, carry):
            for i in range(8):
                b = j * 128 + i * 16
                s16 = src_v[pl.ds(b, 16)]
                d16 = dst_v[pl.ds(b, 16)]
                g0 = plsc.load_gather(s0_v, [s16])
                g1 = plsc.load_gather(s1_v, [d16])
                lg = g0 + g1 + se_v[pl.ds(b, 16)]
                lg = jnp.maximum(lg, lg * 0.2)
                pre = m_v[...] + g1
                gt = jnp.maximum(pre, pre * 0.2)
                ex_v[pl.ds(b, 16)] = jnp.exp(lg - gt)

            @pl.when(j >= 1)
            def _():
                _den_desc(j - 1).wait()

            pltpu.async_copy(ex_v.at[pl.ds(j * 128, 128)],
                             shared_den.at[dst_m.at[j]], dsem, add=True)
            return carry

        lax.fori_loop(0, NB, exbatch, 0)
        _den_desc(NB - 1).wait()

    def _g_issue(j, b, hs_hbm):
        pltpu.async_copy(hs_hbm.at[src_v.at[pl.ds(j * 128, 128)]], rows[b],
                         gsems[b])

    def _g_wait(j, b, hs_hbm):
        pltpu.make_async_copy(hs_hbm.at[src_v.at[pl.ds(j * 128, 128)]],
                              rows[b], gsems[b]).wait()

    def _s_issue(j, b):
        pltpu.async_copy(rows[b], shared_out.at[dst_m.at[j]], ssems[b],
                         add=True)

    def _s_wait(j, b):
        pltpu.make_async_copy(rows[b], shared_out.at[dst_m.at[j]],
                              ssems[b]).wait()

    def _pipe(hs_hbm):
        _g_issue(0, 0, hs_hbm)
        _g_issue(1, 1, hs_hbm)

        def outer(i, carry):
            for b4 in range(4):
                j = i * 4 + b4
                _g_wait(j, b4, hs_hbm)
                rv = rows[b4]

                def scale16(k, c2):
                    ex16 = ex_v[pl.ds(j * 128 + k * 16, 16)]
                    for e in range(16):
                        a = ex16[e]
                        r = k * 16 + e
                        for c in range(HLAT // 16):
                            sl = pl.ds(c * 16, 16)
                            rv[r, sl] = rv[r, sl] * a
                    return c2

                lax.fori_loop(0, 8, scale16, 0)
                _s_issue(j, b4)
                b2 = (b4 + 2) % 4

                @pl.when(j + 2 < NB)
                def _():
                    @pl.when(j >= 2)
                    def __():
                        _s_wait(j - 2, b2)

                    _g_issue(j + 2, b2, hs_hbm)
            return carry

        lax.fori_loop(0, NB // 4, outer, 0)
        for jj in range(NB - 4, NB):
            _s_wait(jj, jj % 4)

    for half in range(2):
        hs_hbm = hslo_hbm if half == 0 else hshi_hbm
        out_hbm = outlo_hbm if half == 0 else outhi_hbm

        _zero_out_slice()
        if half == 0:
            pltpu.sync_copy(zden_v.at[pl.ds(0, NROWS)],
                            shared_den.at[pl.ds(sid * NROWS, NROWS)])
        plsc.subcore_barrier()

        if half == 0:
            _ex_phase()
        _pipe(hs_hbm)
        plsc.subcore_barrier()

        pltpu.sync_copy(shared_out.at[pl.ds(sid * NROWS, NROWS)],
                        out_hbm.at[cid, pl.ds(sid * NROWS, NROWS)])
        if half == 0:
            pltpu.sync_copy(shared_den.at[pl.ds(sid * NROWS, NROWS)],
                            den_hbm.at[cid, pl.ds(sid * NROWS, NROWS)])


_sc_call = functools.partial(
    pl.kernel,
    out_type=(
        jax.ShapeDtypeStruct((NC, NPAD, LAT // 2), jnp.float32),
        jax.ShapeDtypeStruct((NC, NPAD, LAT // 2), jnp.float32),
        jax.ShapeDtypeStruct((NC, NPAD), jnp.float32),
    ),
    mesh=plsc.VectorSubcoreMesh(core_axis_name="c", subcore_axis_name="s"),
    compiler_params=pltpu.CompilerParams(needs_layout_passes=False,
                                         use_tc_tiling_on_sc=False),
    scratch_types=[
        pltpu.VMEM((NPAD,), jnp.float32),
        pltpu.VMEM((NPAD,), jnp.float32),
        pltpu.VMEM((16,), jnp.float32),
        pltpu.VMEM((EW,), jnp.float32),
        pltpu.VMEM((EW,), jnp.int32),
        pltpu.VMEM((EW,), jnp.int32),
        pltpu.VMEM((NB, 128), jnp.int32),
        pltpu.VMEM((EW,), jnp.float32),
        pltpu.VMEM((128, LAT // 2), jnp.float32),
        pltpu.VMEM((128, LAT // 2), jnp.float32),
        pltpu.VMEM((128, LAT // 2), jnp.float32),
        pltpu.VMEM((128, LAT // 2), jnp.float32),
        pltpu.VMEM((128, LAT // 2), jnp.float32),
        pltpu.VMEM((640,), jnp.float32),
        pltpu.VMEM_SHARED((NPAD, LAT // 2), jnp.float32),
        pltpu.VMEM_SHARED((NPAD,), jnp.float32),
        pltpu.SemaphoreType.DMA,
        pltpu.SemaphoreType.DMA,
        pltpu.SemaphoreType.DMA,
        pltpu.SemaphoreType.DMA,
        pltpu.SemaphoreType.DMA,
        pltpu.SemaphoreType.DMA,
        pltpu.SemaphoreType.DMA,
        pltpu.SemaphoreType.DMA,
        pltpu.SemaphoreType.DMA,
    ],
)(_sc_body)


FBLK = 1264


def _mlp_body(plo_ref, phi_ref, den_ref, w1_ref, b1_ref, w2_ref, b2_ref,
              av_ref, v_ref, minv_ref, m1_ref):
    den = den_ref[0] + den_ref[1] + 1e-9
    o = jnp.concatenate([plo_ref[0] + plo_ref[1], phi_ref[0] + phi_ref[1]],
                        axis=1) / den
    h = jnp.where(o > 0, o, (jnp.exp(o) - 1.0))
    t1 = jnp.maximum(
        jnp.dot(h, w1_ref[...], precision=_HI,
                preferred_element_type=jnp.float32) + b1_ref[...], 0.0)
    v = jnp.dot(t1, w2_ref[...], precision=_HI,
                preferred_element_type=jnp.float32) + b2_ref[...]
    v_ref[...] = v
    i = pl.program_id(0)
    rows = lax.broadcasted_iota(jnp.int32, (FBLK, 1), 0) + i * FBLK
    mask = rows < NN
    av = av_ref[...]
    bmin = jnp.full((1, 1), jnp.min(jnp.where(mask, v, jnp.inf)))
    bm1 = jnp.full((1, 1),
                   jnp.max(jnp.where(mask & (av > 0), v, -jnp.inf)))
    pmin = jnp.where(i == 0, jnp.full((1, 1), jnp.inf), minv_ref[...])
    pm1 = jnp.where(i == 0, jnp.full((1, 1), -jnp.inf), m1_ref[...])
    minv_ref[...] = jnp.minimum(pmin, bmin)
    m1_ref[...] = jnp.maximum(pm1, bm1)


def _mlp_call(plo, phi, den, W1, b1, W2, b2, av):
    return pl.pallas_call(
        _mlp_body,
        grid=(NPAD // FBLK,),
        in_specs=[
            pl.BlockSpec((NC, FBLK, LAT // 2), lambda i: (0, i, 0)),
            pl.BlockSpec((NC, FBLK, LAT // 2), lambda i: (0, i, 0)),
            pl.BlockSpec((NC, FBLK, 1), lambda i: (0, i, 0)),
            pl.BlockSpec((LAT, HID), lambda i: (0, 0)),
            pl.BlockSpec((1, HID), lambda i: (0, 0)),
            pl.BlockSpec((HID, 1), lambda i: (0, 0)),
            pl.BlockSpec((1, 1), lambda i: (0, 0)),
            pl.BlockSpec((FBLK, 1), lambda i: (i, 0)),
        ],
        out_specs=[
            pl.BlockSpec((FBLK, 1), lambda i: (i, 0)),
            pl.BlockSpec((1, 1), lambda i: (0, 0)),
            pl.BlockSpec((1, 1), lambda i: (0, 0)),
        ],
        out_shape=[
            jax.ShapeDtypeStruct((NPAD, 1), jnp.float32),
            jax.ShapeDtypeStruct((1, 1), jnp.float32),
            jax.ShapeDtypeStruct((1, 1), jnp.float32),
        ],
    )(plo, phi, den, W1, b1, W2, b2, av)


def _soft_body(v_ref, av_ref, cz_ref, num_ref, ssum_ref, nsum_ref):
    i = pl.program_id(0)
    rows = lax.broadcasted_iota(jnp.int32, (FBLK, 1), 0) + i * FBLK
    mask = rows < NN
    av = av_ref[...]
    z2 = (v_ref[...] + cz_ref[...]) * av
    sv = jnp.where(mask, jnp.exp(z2), 0.0)
    num = sv * av
    num_ref[...] = num
    ps = jnp.where(i == 0, jnp.zeros((1, 1)), ssum_ref[...])
    pn = jnp.where(i == 0, jnp.zeros((1, 1)), nsum_ref[...])
    ssum_ref[...] = ps + jnp.full((1, 1), jnp.sum(sv))
    nsum_ref[...] = pn + jnp.full((1, 1), jnp.sum(num))


def _soft_call(v, av, cz):
    return pl.pallas_call(
        _soft_body,
        grid=(NPAD // FBLK,),
        in_specs=[
            pl.BlockSpec((FBLK, 1), lambda i: (i, 0)),
            pl.BlockSpec((FBLK, 1), lambda i: (i, 0)),
            pl.BlockSpec((1, 1), lambda i: (0, 0)),
        ],
        out_specs=[
            pl.BlockSpec((FBLK, 1), lambda i: (i, 0)),
            pl.BlockSpec((1, 1), lambda i: (0, 0)),
            pl.BlockSpec((1, 1), lambda i: (0, 0)),
        ],
        out_shape=[
            jax.ShapeDtypeStruct((NPAD, 1), jnp.float32),
            jax.ShapeDtypeStruct((1, 1), jnp.float32),
            jax.ShapeDtypeStruct((1, 1), jnp.float32),
        ],
    )(v, av, cz)


def _scale_body(num_ref, inv_ref, out_ref):
    out_ref[...] = num_ref[...] * inv_ref[...]


def _scale_call(num, inv):
    return pl.pallas_call(
        _scale_body,
        grid=(NPAD // FBLK,),
        in_specs=[
            pl.BlockSpec((FBLK, 1), lambda i: (i, 0)),
            pl.BlockSpec((1, 1), lambda i: (0, 0)),
        ],
        out_specs=pl.BlockSpec((FBLK, 1), lambda i: (i, 0)),
        out_shape=jax.ShapeDtypeStruct((NPAD, 1), jnp.float32),
    )(num, inv)


def kernel(x, edge_index, edge_attr, available, params):
    pad_e = EPAD - EE
    src_p = jnp.concatenate(
        [edge_index[0], jnp.full((pad_e,), NN, jnp.int32)])
    dst_p = jnp.concatenate(
        [edge_index[1], jnp.full((pad_e,), NN, jnp.int32)])
    dst2d = jnp.concatenate(
        [edge_index[1].reshape(EE // 128, 128),
         jnp.full((pad_e // 128, 128), NN, jnp.int32)], axis=0)
    x_p = jnp.pad(x, ((0, NPAD - NN), (0, 0)))
    eT_cur = jnp.pad(edge_attr, ((0, pad_e), (0, 0))).T
    av_p = jnp.pad(available, (0, NPAD - NN)).reshape(NPAD, 1)

    plo = phi = den = None
    for l in range(5):
        a = params[f'a{l}']
        a01 = jnp.stack([a[:LAT], a[LAT:2 * LAT]], axis=1)
        a2r = a[2 * LAT:].reshape(1, LAT)
        enT, seT, semax = _edge_call(eT_cur, params[f'We{l}'].T, a2r)
        if l == 0:
            hslo, hshi, s0, s1, s0max = _node0_call(x_p, params[f'Wn{l}'],
                                                    a01)
        else:
            hslo, hshi, s0, s1, s0max = _nodec_call(plo, phi, den,
                                                    params[f'Wn{l}'], a01)
        m16 = jnp.full((16,), s0max[0, 0] + semax[0, 0], jnp.float32)
        plo, phi, den2 = _sc_call(hslo, hshi, s0.reshape(NPAD),
                                  s1.reshape(NPAD), m16, seT.reshape(EPAD),
                                  src_p, dst_p, dst2d)
        den = den2.reshape(NC, NPAD, 1)
        eT_cur = enT

    v, minv, m1 = _mlp_call(plo, phi, den, params['W1'],
                            params['b1'].reshape(1, HID), params['W2'],
                            params['b2'].reshape(1, 1), av_p)
    c = jnp.abs(minv[0, 0])
    zm = jnp.maximum(0.0, m1[0, 0] + c)
    cz = jnp.full((1, 1), c - zm, jnp.float32)
    num, ssum, nsum = _soft_call(v, av_p, cz)
    inv = (1.0 / (nsum + 1e-13 * ssum)).reshape(1, 1)
    out = _scale_call(num, inv)
    return out.reshape(NPAD)[:NN]


# speedup vs baseline: 15.8699x; 1.0193x over previous
import functools

import jax
import jax.numpy as jnp
from jax import lax
from jax.experimental import pallas as pl
from jax.experimental.pallas import tpu as pltpu
from jax.experimental.pallas import tpu_sc as plsc

NN = 10000
EE = 320000
LAT = 64
HID = 128
NPAD = 10112
EPAD = 327680
NC, NS = 2, 16
NW = NC * NS
EW = EPAD // NW
NB = EW // 128
NROWS = NPAD // NS

_HI = lax.Precision.HIGHEST


def _edge_body(eT_ref, weT_ref, a2r_ref, enT_ref, seT_ref, semax_ref):
    etT = jnp.dot(weT_ref[...], eT_ref[...], precision=_HI,
                  preferred_element_type=jnp.float32)
    enT_ref[...] = jnp.where(etT > 0, etT, (jnp.exp(etT) - 1.0))
    sev = jnp.dot(a2r_ref[...], etT, precision=_HI,
                  preferred_element_type=jnp.float32)
    seT_ref[...] = sev
    i = pl.program_id(0)
    prev = jnp.where(i == 0, jnp.full((1, 1), -jnp.inf), semax_ref[...])
    semax_ref[...] = jnp.maximum(prev, jnp.full((1, 1), jnp.max(sev)))


def _edge_call(eT, WeT, a2r):
    rb = 4096
    de = eT.shape[0]
    return pl.pallas_call(
        _edge_body,
        grid=(EPAD // rb,),
        in_specs=[
            pl.BlockSpec((de, rb), lambda i: (0, i)),
            pl.BlockSpec((LAT, de), lambda i: (0, 0)),
            pl.BlockSpec((1, LAT), lambda i: (0, 0)),
        ],
        out_specs=[
            pl.BlockSpec((LAT, rb), lambda i: (0, i)),
            pl.BlockSpec((1, rb), lambda i: (0, i)),
            pl.BlockSpec((1, 1), lambda i: (0, 0)),
        ],
        out_shape=[
            jax.ShapeDtypeStruct((LAT, EPAD), jnp.float32),
            jax.ShapeDtypeStruct((1, EPAD), jnp.float32),
            jax.ShapeDtypeStruct((1, 1), jnp.float32),
        ],
    )(eT, WeT, a2r)


NBLK = 1264


def _node_tail(h, wn_ref, a01_ref, hslo_ref, hshi_ref, s0_ref, s1_ref,
               s0max_ref):
    hs = jnp.dot(h, wn_ref[...], precision=_HI,
                 preferred_element_type=jnp.float32)
    hslo_ref[...] = hs[:, :LAT // 2]
    hshi_ref[...] = hs[:, LAT // 2:]
    s = jnp.dot(hs, a01_ref[...], precision=_HI,
                preferred_element_type=jnp.float32)
    s0_ref[...] = s[:, 0:1]
    s1_ref[...] = s[:, 1:2]
    i = pl.program_id(0)
    prev = jnp.where(i == 0, jnp.full((1, 1), -jnp.inf), s0max_ref[...])
    s0max_ref[...] = jnp.maximum(prev, jnp.full((1, 1), jnp.max(s[:, 0])))


def _node0_body(x_ref, wn_ref, a01_ref, hslo_ref, hshi_ref, s0_ref, s1_ref,
                s0max_ref):
    _node_tail(x_ref[...], wn_ref, a01_ref, hslo_ref, hshi_ref, s0_ref,
               s1_ref, s0max_ref)


def _nodec_body(plo_ref, phi_ref, den_ref, wn_ref, a01_ref, hslo_ref,
                hshi_ref, s0_ref, s1_ref, s0max_ref):
    den = den_ref[0] + den_ref[1] + 1e-9
    o = jnp.concatenate([plo_ref[0] + plo_ref[1], phi_ref[0] + phi_ref[1]],
                        axis=1) / den
    h = jnp.where(o > 0, o, (jnp.exp(o) - 1.0))
    _node_tail(h, wn_ref, a01_ref, hslo_ref, hshi_ref, s0_ref, s1_ref,
               s0max_ref)


_NODE_OUT = [
    jax.ShapeDtypeStruct((NPAD, LAT // 2), jnp.float32),
    jax.ShapeDtypeStruct((NPAD, LAT // 2), jnp.float32),
    jax.ShapeDtypeStruct((NPAD, 1), jnp.float32),
    jax.ShapeDtypeStruct((NPAD, 1), jnp.float32),
    jax.ShapeDtypeStruct((1, 1), jnp.float32),
]

_NODE_OUT_SPECS = [
    pl.BlockSpec((NBLK, LAT // 2), lambda i: (i, 0)),
    pl.BlockSpec((NBLK, LAT // 2), lambda i: (i, 0)),
    pl.BlockSpec((NBLK, 1), lambda i: (i, 0)),
    pl.BlockSpec((NBLK, 1), lambda i: (i, 0)),
    pl.BlockSpec((1, 1), lambda i: (0, 0)),
]


def _node0_call(x, Wn, a01):
    dn = x.shape[1]
    return pl.pallas_call(
        _node0_body,
        grid=(NPAD // NBLK,),
        in_specs=[
            pl.BlockSpec((NBLK, dn), lambda i: (i, 0)),
            pl.BlockSpec((dn, LAT), lambda i: (0, 0)),
            pl.BlockSpec((LAT, 2), lambda i: (0, 0)),
        ],
        out_specs=_NODE_OUT_SPECS,
        out_shape=_NODE_OUT,
    )(x, Wn, a01)


def _nodec_call(plo, phi, den, Wn, a01):
    return pl.pallas_call(
        _nodec_body,
        grid=(NPAD // NBLK,),
        in_specs=[
            pl.BlockSpec((NC, NBLK, LAT // 2), lambda i: (0, i, 0)),
            pl.BlockSpec((NC, NBLK, LAT // 2), lambda i: (0, i, 0)),
            pl.BlockSpec((NC, NBLK, 1), lambda i: (0, i, 0)),
            pl.BlockSpec((LAT, LAT), lambda i: (0, 0)),
            pl.BlockSpec((LAT, 2), lambda i: (0, 0)),
        ],
        out_specs=_NODE_OUT_SPECS,
        out_shape=_NODE_OUT,
    )(plo, phi, den, Wn, a01)


def _sc_body(hslo_hbm, hshi_hbm, s0_hbm, s1_hbm, m_hbm, se_hbm, src_hbm,
             dst_hbm, dst2d_hbm, outlo_hbm, outhi_hbm, den_hbm, s0_v, s1_v,
             m_v, se_v, src_v, dst_v, dst_m, ex_v, r0, r1, r2, r3, zrow_v,
             zden_v, shared_out, shared_den, gs0, gs1, gs2, gs3, ss0, ss1,
             ss2, ss3, dsem):
    cid = lax.axis_index("c")
    sid = lax.axis_index("s")
    wid = sid * NC + cid
    ebase = wid * EW
    rbase = wid * NB
    HLAT = LAT // 2
    rows = (r0, r1, r2, r3)
    gsems = (gs0, gs1, gs2, gs3)
    ssems = (ss0, ss1, ss2, ss3)

    pltpu.sync_copy(s0_hbm, s0_v)
    pltpu.sync_copy(s1_hbm, s1_v)
    pltpu.sync_copy(m_hbm, m_v)
    pltpu.sync_copy(se_hbm.at[pl.ds(ebase, EW)], se_v)
    pltpu.sync_copy(src_hbm.at[pl.ds(ebase, EW)], src_v)
    pltpu.sync_copy(dst_hbm.at[pl.ds(ebase, EW)], dst_v)
    pltpu.sync_copy(dst2d_hbm.at[pl.ds(rbase, NB)], dst_m)

    zeros16 = jnp.zeros((16,), jnp.float32)

    def zrow(r, _):
        for c in range(HLAT // 16):
            zrow_v[r, pl.ds(c * 16, 16)] = zeros16
        return 0

    lax.fori_loop(0, 128, zrow, 0)

    def zden(r, _):
        zden_v[pl.ds(r * 16, 16)] = zeros16
        return 0

    lax.fori_loop(0, 640 // 16, zden, 0)

    def _zero_out_slice():
        for r in range(NROWS // 128):
            pltpu.sync_copy(
                zrow_v, shared_out.at[pl.ds(sid * NROWS + r * 128, 128)])
        _rem = NROWS % 128
        if _rem:
            pltpu.sync_copy(
                zrow_v.at[pl.ds(0, _rem)],
                shared_out.at[pl.ds(sid * NROWS + (NROWS // 128) * 128,
                                    _rem)])

    def _den_desc(j):
        return pltpu.make_async_copy(ex_v.at[pl.ds(j * 128, 128)],
                                     shared_den.at[dst_m.at[j]], dsem)

    def _g_issue(j, b, hs_hbm):
        pltpu.async_copy(hs_hbm.at[src_v.at[pl.ds(j * 128, 128)]], rows[b],
                         gsems[b])

    def _g_wait(j, b, hs_hbm):
        pltpu.make_async_copy(hs_hbm.at[src_v.at[pl.ds(j * 128, 128)]],
                              rows[b], gsems[b]).wait()

    def _s_issue(j, b):
        pltpu.async_copy(rows[b], shared_out.at[dst_m.at[j]], ssems[b],
                         add=True)

    def _s_wait(j, b):
        pltpu.make_async_copy(rows[b], shared_out.at[dst_m.at[j]],
                              ssems[b]).wait()

    def _pipe(hs_hbm, with_ex):
        _g_issue(0, 0, hs_hbm)
        _g_issue(1, 1, hs_hbm)

        def outer(i, carry):
            for b4 in range(4):
                j = i * 4 + b4
                if with_ex:
                    for i8 in range(8):
                        b = j * 128 + i8 * 16
                        s16 = src_v[pl.ds(b, 16)]
                        d16 = dst_v[pl.ds(b, 16)]
                        g0 = plsc.load_gather(s0_v, [s16])
                        g1 = plsc.load_gather(s1_v, [d16])
                        lg = g0 + g1 + se_v[pl.ds(b, 16)]
                        lg = jnp.maximum(lg, lg * 0.2)
                        pre = m_v[...] + g1
                        gt = jnp.maximum(pre, pre * 0.2)
                        ex_v[pl.ds(b, 16)] = jnp.exp(lg - gt)

                    @pl.when(j >= 1)
                    def _dw():
                        _den_desc(j - 1).wait()

                    pltpu.async_copy(ex_v.at[pl.ds(j * 128, 128)],
                                     shared_den.at[dst_m.at[j]], dsem,
                                     add=True)
                _g_wait(j, b4, hs_hbm)
                rv = rows[b4]

                def scale16(k, c2):
                    ex16 = ex_v[pl.ds(j * 128 + k * 16, 16)]
                    for e in range(16):
                        a = ex16[e]
                        r = k * 16 + e
                        for c in range(HLAT // 16):
                            sl = pl.ds(c * 16, 16)
                            rv[r, sl] = rv[r, sl] * a
                    return c2

                lax.fori_loop(0, 8, scale16, 0)
                _s_issue(j, b4)
                b2 = (b4 + 2) % 4

                @pl.when(j + 2 < NB)
                def _():
                    @pl.when(j >= 2)
                    def __():
                        _s_wait(j - 2, b2)

                    _g_issue(j + 2, b2, hs_hbm)
            return carry

        lax.fori_loop(0, NB // 4, outer, 0)
        if with_ex:
            _den_desc(NB - 1).wait()
        for jj in range(NB - 4, NB):
            _s_wait(jj, jj % 4)

    for half in range(2):
        hs_hbm = hslo_hbm if half == 0 else hshi_hbm
        out_hbm = outlo_hbm if half == 0 else outhi_hbm

        _zero_out_slice()
        if half == 0:
            pltpu.sync_copy(zden_v.at[pl.ds(0, NROWS)],
                            shared_den.at[pl.ds(sid * NROWS, NROWS)])
        plsc.subcore_barrier()

        _pipe(hs_hbm, with_ex=(half == 0))
        plsc.subcore_barrier()

        pltpu.sync_copy(shared_out.at[pl.ds(sid * NROWS, NROWS)],
                        out_hbm.at[cid, pl.ds(sid * NROWS, NROWS)])
        if half == 0:
            pltpu.sync_copy(shared_den.at[pl.ds(sid * NROWS, NROWS)],
                            den_hbm.at[cid, pl.ds(sid * NROWS, NROWS)])


_sc_call = functools.partial(
    pl.kernel,
    out_type=(
        jax.ShapeDtypeStruct((NC, NPAD, LAT // 2), jnp.float32),
        jax.ShapeDtypeStruct((NC, NPAD, LAT // 2), jnp.float32),
        jax.ShapeDtypeStruct((NC, NPAD), jnp.float32),
    ),
    mesh=plsc.VectorSubcoreMesh(core_axis_name="c", subcore_axis_name="s"),
    compiler_params=pltpu.CompilerParams(needs_layout_passes=False,
                                         use_tc_tiling_on_sc=False),
    scratch_types=[
        pltpu.VMEM((NPAD,), jnp.float32),
        pltpu.VMEM((NPAD,), jnp.float32),
        pltpu.VMEM((16,), jnp.float32),
        pltpu.VMEM((EW,), jnp.float32),
        pltpu.VMEM((EW,), jnp.int32),
        pltpu.VMEM((EW,), jnp.int32),
        pltpu.VMEM((NB, 128), jnp.int32),
        pltpu.VMEM((EW,), jnp.float32),
        pltpu.VMEM((128, LAT // 2), jnp.float32),
        pltpu.VMEM((128, LAT // 2), jnp.float32),
        pltpu.VMEM((128, LAT // 2), jnp.float32),
        pltpu.VMEM((128, LAT // 2), jnp.float32),
        pltpu.VMEM((128, LAT // 2), jnp.float32),
        pltpu.VMEM((640,), jnp.float32),
        pltpu.VMEM_SHARED((NPAD, LAT // 2), jnp.float32),
        pltpu.VMEM_SHARED((NPAD,), jnp.float32),
        pltpu.SemaphoreType.DMA,
        pltpu.SemaphoreType.DMA,
        pltpu.SemaphoreType.DMA,
        pltpu.SemaphoreType.DMA,
        pltpu.SemaphoreType.DMA,
        pltpu.SemaphoreType.DMA,
        pltpu.SemaphoreType.DMA,
        pltpu.SemaphoreType.DMA,
        pltpu.SemaphoreType.DMA,
    ],
)(_sc_body)


FBLK = 1264


def _mlp_body(plo_ref, phi_ref, den_ref, w1_ref, b1_ref, w2_ref, b2_ref,
              av_ref, v_ref, minv_ref, m1_ref):
    den = den_ref[0] + den_ref[1] + 1e-9
    o = jnp.concatenate([plo_ref[0] + plo_ref[1], phi_ref[0] + phi_ref[1]],
                        axis=1) / den
    h = jnp.where(o > 0, o, (jnp.exp(o) - 1.0))
    t1 = jnp.maximum(
        jnp.dot(h, w1_ref[...], precision=_HI,
                preferred_element_type=jnp.float32) + b1_ref[...], 0.0)
    v = jnp.dot(t1, w2_ref[...], precision=_HI,
                preferred_element_type=jnp.float32) + b2_ref[...]
    v_ref[...] = v
    i = pl.program_id(0)
    rows = lax.broadcasted_iota(jnp.int32, (FBLK, 1), 0) + i * FBLK
    mask = rows < NN
    av = av_ref[...]
    bmin = jnp.full((1, 1), jnp.min(jnp.where(mask, v, jnp.inf)))
    bm1 = jnp.full((1, 1),
                   jnp.max(jnp.where(mask & (av > 0), v, -jnp.inf)))
    pmin = jnp.where(i == 0, jnp.full((1, 1), jnp.inf), minv_ref[...])
    pm1 = jnp.where(i == 0, jnp.full((1, 1), -jnp.inf), m1_ref[...])
    minv_ref[...] = jnp.minimum(pmin, bmin)
    m1_ref[...] = jnp.maximum(pm1, bm1)


def _mlp_call(plo, phi, den, W1, b1, W2, b2, av):
    return pl.pallas_call(
        _mlp_body,
        grid=(NPAD // FBLK,),
        in_specs=[
            pl.BlockSpec((NC, FBLK, LAT // 2), lambda i: (0, i, 0)),
            pl.BlockSpec((NC, FBLK, LAT // 2), lambda i: (0, i, 0)),
            pl.BlockSpec((NC, FBLK, 1), lambda i: (0, i, 0)),
            pl.BlockSpec((LAT, HID), lambda i: (0, 0)),
            pl.BlockSpec((1, HID), lambda i: (0, 0)),
            pl.BlockSpec((HID, 1), lambda i: (0, 0)),
            pl.BlockSpec((1, 1), lambda i: (0, 0)),
            pl.BlockSpec((FBLK, 1), lambda i: (i, 0)),
        ],
        out_specs=[
            pl.BlockSpec((FBLK, 1), lambda i: (i, 0)),
            pl.BlockSpec((1, 1), lambda i: (0, 0)),
            pl.BlockSpec((1, 1), lambda i: (0, 0)),
        ],
        out_shape=[
            jax.ShapeDtypeStruct((NPAD, 1), jnp.float32),
            jax.ShapeDtypeStruct((1, 1), jnp.float32),
            jax.ShapeDtypeStruct((1, 1), jnp.float32),
        ],
    )(plo, phi, den, W1, b1, W2, b2, av)


def _soft_body(v_ref, av_ref, cz_ref, num_ref, ssum_ref, nsum_ref):
    i = pl.program_id(0)
    rows = lax.broadcasted_iota(jnp.int32, (FBLK, 1), 0) + i * FBLK
    mask = rows < NN
    av = av_ref[...]
    z2 = (v_ref[...] + cz_ref[...]) * av
    sv = jnp.where(mask, jnp.exp(z2), 0.0)
    num = sv * av
    num_ref[...] = num
    ps = jnp.where(i == 0, jnp.zeros((1, 1)), ssum_ref[...])
    pn = jnp.where(i == 0, jnp.zeros((1, 1)), nsum_ref[...])
    ssum_ref[...] = ps + jnp.full((1, 1), jnp.sum(sv))
    nsum_ref[...] = pn + jnp.full((1, 1), jnp.sum(num))


def _soft_call(v, av, cz):
    return pl.pallas_call(
        _soft_body,
        grid=(NPAD // FBLK,),
        in_specs=[
            pl.BlockSpec((FBLK, 1), lambda i: (i, 0)),
            pl.BlockSpec((FBLK, 1), lambda i: (i, 0)),
            pl.BlockSpec((1, 1), lambda i: (0, 0)),
        ],
        out_specs=[
            pl.BlockSpec((FBLK, 1), lambda i: (i, 0)),
            pl.BlockSpec((1, 1), lambda i: (0, 0)),
            pl.BlockSpec((1, 1), lambda i: (0, 0)),
        ],
        out_shape=[
            jax.ShapeDtypeStruct((NPAD, 1), jnp.float32),
            jax.ShapeDtypeStruct((1, 1), jnp.float32),
            jax.ShapeDtypeStruct((1, 1), jnp.float32),
        ],
    )(v, av, cz)


def _scale_body(num_ref, inv_ref, out_ref):
    out_ref[...] = num_ref[...] * inv_ref[...]


def _scale_call(num, inv):
    return pl.pallas_call(
        _scale_body,
        grid=(NPAD // FBLK,),
        in_specs=[
            pl.BlockSpec((FBLK, 1), lambda i: (i, 0)),
            pl.BlockSpec((1, 1), lambda i: (0, 0)),
        ],
        out_specs=pl.BlockSpec((FBLK, 1), lambda i: (i, 0)),
        out_shape=jax.ShapeDtypeStruct((NPAD, 1), jnp.float32),
    )(num, inv)


def kernel(x, edge_index, edge_attr, available, params):
    pad_e = EPAD - EE
    src_p = jnp.concatenate(
        [edge_index[0], jnp.full((pad_e,), NN, jnp.int32)])
    dst_p = jnp.concatenate(
        [edge_index[1], jnp.full((pad_e,), NN, jnp.int32)])
    dst2d = jnp.concatenate(
        [edge_index[1].reshape(EE // 128, 128),
         jnp.full((pad_e // 128, 128), NN, jnp.int32)], axis=0)
    x_p = jnp.pad(x, ((0, NPAD - NN), (0, 0)))
    eT_cur = jnp.pad(edge_attr, ((0, pad_e), (0, 0))).T
    av_p = jnp.pad(available, (0, NPAD - NN)).reshape(NPAD, 1)

    plo = phi = den = None
    for l in range(5):
        a = params[f'a{l}']
        a01 = jnp.stack([a[:LAT], a[LAT:2 * LAT]], axis=1)
        a2r = a[2 * LAT:].reshape(1, LAT)
        enT, seT, semax = _edge_call(eT_cur, params[f'We{l}'].T, a2r)
        if l == 0:
            hslo, hshi, s0, s1, s0max = _node0_call(x_p, params[f'Wn{l}'],
                                                    a01)
        else:
            hslo, hshi, s0, s1, s0max = _nodec_call(plo, phi, den,
                                                    params[f'Wn{l}'], a01)
        m16 = jnp.full((16,), s0max[0, 0] + semax[0, 0], jnp.float32)
        plo, phi, den2 = _sc_call(hslo, hshi, s0.reshape(NPAD),
                                  s1.reshape(NPAD), m16, seT.reshape(EPAD),
                                  src_p, dst_p, dst2d)
        den = den2.reshape(NC, NPAD, 1)
        eT_cur = enT

    v, minv, m1 = _mlp_call(plo, phi, den, params['W1'],
                            params['b1'].reshape(1, HID), params['W2'],
                            params['b2'].reshape(1, 1), av_p)
    c = jnp.abs(minv[0, 0])
    zm = jnp.maximum(0.0, m1[0, 0] + c)
    cz = jnp.full((1, 1), c - zm, jnp.float32)
    num, ssum, nsum = _soft_call(v, av_p, cz)
    inv = (1.0 / (nsum + 1e-13 * ssum)).reshape(1, 1)
    out = _scale_call(num, inv)
    return out.reshape(NPAD)[:NN]
